# Initial kernel scaffold; baseline (speedup 1.0000x reference)
import jax
import jax.numpy as jnp
from jax.experimental import pallas as pl


def kernel(x, edge_index, W1, b1, W2, b2):
    raise NotImplementedError("write your pallas kernel here")


# speedup vs baseline: 7.8559x; 7.8559x over previous
import functools

import jax
import jax.numpy as jnp
from jax import lax
from jax.experimental import pallas as pl
from jax.experimental.pallas import tpu as pltpu
from jax.experimental.pallas import tpu_sc as plsc

N = 10000
E = 320000
D = 128

NC = 2
NS = 16
NW = NC * NS
NPAD = 10240
RPT = NPAD // NS
C = 128
EPAD = 327680
NCHUNK = EPAD // (NW * C)

_mesh = plsc.VectorSubcoreMesh(core_axis_name="c", subcore_axis_name="s")


@functools.partial(
    pl.kernel,
    out_type=jax.ShapeDtypeStruct((NC, NPAD), jnp.float32),
    mesh=_mesh,
    scratch_types=[
        pltpu.VMEM((NCHUNK, C), jnp.int32),
        pltpu.VMEM((C,), jnp.float32),
        pltpu.VMEM((RPT,), jnp.float32),
        pltpu.VMEM_SHARED((NPAD,), jnp.float32),
        pltpu.SemaphoreType.DMA,
    ],
)
def _sc_degree(dst_hbm, out_hbm, dst_v, ones_v, zero_v, deg_sh, sem):
    cid = lax.axis_index("c")
    sid = lax.axis_index("s")
    wid = cid * NS + sid

    pltpu.async_copy(dst_hbm.at[wid], dst_v, sem).wait()

    def fill(i, _):
        ones_v[pl.ds(i * 16, 16)] = jnp.ones((16,), jnp.float32)
        return 0
    lax.fori_loop(0, C // 16, fill, 0)

    def fillz(i, _):
        zero_v[pl.ds(i * 16, 16)] = jnp.zeros((16,), jnp.float32)
        return 0
    lax.fori_loop(0, RPT // 16, fillz, 0)

    pltpu.sync_copy(zero_v, deg_sh.at[pl.ds(sid * RPT, RPT)])
    plsc.subcore_barrier()

    def body(j, _):
        pltpu.sync_copy(ones_v, deg_sh.at[dst_v.at[j]], add=True)
        return 0
    lax.fori_loop(0, NCHUNK, body, 0)

    plsc.subcore_barrier()
    pltpu.sync_copy(deg_sh.at[pl.ds(sid * RPT, RPT)],
                    out_hbm.at[cid, pl.ds(sid * RPT, RPT)])


@functools.partial(
    pl.kernel,
    out_type=jax.ShapeDtypeStruct((NC, NPAD, D), jnp.float32),
    mesh=_mesh,
    scratch_types=[
        pltpu.VMEM((NCHUNK, C), jnp.int32),
        pltpu.VMEM((NCHUNK, C), jnp.int32),
        pltpu.VMEM((C, D), jnp.float32),
        pltpu.VMEM_SHARED((NPAD, D), jnp.float32),
        pltpu.SemaphoreType.DMA,
    ],
)
def _sc_gather_scatter(h_hbm, src_hbm, dst_hbm, out_hbm,
                       src_v, dst_v, buf, agg_sh, sem):
    cid = lax.axis_index("c")
    sid = lax.axis_index("s")
    wid = cid * NS + sid

    pltpu.async_copy(src_hbm.at[wid], src_v, sem).wait()
    pltpu.async_copy(dst_hbm.at[wid], dst_v, sem).wait()

    def fillz(i, _):
        for l in range(D // 16):
            buf[i, pl.ds(l * 16, 16)] = jnp.zeros((16,), jnp.float32)
        return 0
    lax.fori_loop(0, C, fillz, 0)
    for k in range(RPT // C):
        pltpu.sync_copy(buf, agg_sh.at[pl.ds(sid * RPT + k * C, C)])
    plsc.subcore_barrier()

    def body(j, _):
        pltpu.async_copy(h_hbm.at[src_v.at[j]], buf, sem).wait()
        pltpu.sync_copy(buf, agg_sh.at[dst_v.at[j]], add=True)
        return 0
    lax.fori_loop(0, NCHUNK, body, 0)

    plsc.subcore_barrier()
    pltpu.sync_copy(agg_sh.at[pl.ds(sid * RPT, RPT)],
                    out_hbm.at[cid, pl.ds(sid * RPT, RPT)])


BM = 1280


def _tc1_body(x_ref, w_ref, degt_ref, dinv_ref, hp_ref):
    deg = 1.0 + degt_ref[:, 0:1] + degt_ref[:, 1:2]
    dinv = lax.rsqrt(deg)
    dinv_ref[...] = dinv
    hp_ref[...] = jnp.dot(x_ref[...], w_ref[...],
                          preferred_element_type=jnp.float32) * dinv


def _tc2_body(s_ref, hp_ref, dinv_ref, b_ref, w_ref, hp2_ref):
    s = s_ref[0] + s_ref[1]
    dinv = dinv_ref[...]
    z = jnp.maximum(dinv * (s + hp_ref[...]) + b_ref[...], 0.0)
    hp2_ref[...] = jnp.dot(z, w_ref[...],
                           preferred_element_type=jnp.float32) * dinv


def _tc3_body(s_ref, hp_ref, dinv_ref, b_ref, out_ref):
    i = pl.program_id(0)
    s = s_ref[0] + s_ref[1]
    z = jnp.maximum(dinv_ref[...] * (s + hp_ref[...]) + b_ref[...], 0.0)
    rows = lax.broadcasted_iota(jnp.int32, (BM, D), 0) + i * BM
    z = jnp.where(rows < N, z, 0.0)

    @pl.when(i == 0)
    def _():
        out_ref[...] = jnp.zeros_like(out_ref)

    out_ref[...] += jnp.sum(z, axis=0, keepdims=True) * (1.0 / N)


def _tc1(xp, w1, degt):
    return pl.pallas_call(
        _tc1_body,
        grid=(NPAD // BM,),
        in_specs=[
            pl.BlockSpec((BM, D), lambda i: (i, 0)),
            pl.BlockSpec((D, D), lambda i: (0, 0)),
            pl.BlockSpec((BM, NC), lambda i: (i, 0)),
        ],
        out_specs=[
            pl.BlockSpec((BM, 1), lambda i: (i, 0)),
            pl.BlockSpec((BM, D), lambda i: (i, 0)),
        ],
        out_shape=[
            jax.ShapeDtypeStruct((NPAD, 1), jnp.float32),
            jax.ShapeDtypeStruct((NPAD, D), jnp.float32),
        ],
    )(xp, w1, degt)


def _tc2(s, hp, dinv, b, w):
    return pl.pallas_call(
        _tc2_body,
        grid=(NPAD // BM,),
        in_specs=[
            pl.BlockSpec((NC, BM, D), lambda i: (0, i, 0)),
            pl.BlockSpec((BM, D), lambda i: (i, 0)),
            pl.BlockSpec((BM, 1), lambda i: (i, 0)),
            pl.BlockSpec((1, D), lambda i: (0, 0)),
            pl.BlockSpec((D, D), lambda i: (0, 0)),
        ],
        out_specs=pl.BlockSpec((BM, D), lambda i: (i, 0)),
        out_shape=jax.ShapeDtypeStruct((NPAD, D), jnp.float32),
    )(s, hp, dinv, b, w)


def _tc3(s, hp, dinv, b):
    return pl.pallas_call(
        _tc3_body,
        grid=(NPAD // BM,),
        in_specs=[
            pl.BlockSpec((NC, BM, D), lambda i: (0, i, 0)),
            pl.BlockSpec((BM, D), lambda i: (i, 0)),
            pl.BlockSpec((BM, 1), lambda i: (i, 0)),
            pl.BlockSpec((1, D), lambda i: (0, 0)),
        ],
        out_specs=pl.BlockSpec((1, D), lambda i: (0, 0)),
        out_shape=jax.ShapeDtypeStruct((1, D), jnp.float32),
    )(s, hp, dinv, b)


def kernel(x, edge_index, W1, b1, W2, b2):
    src = edge_index[0]
    dst = edge_index[1]
    srcp = jnp.concatenate(
        [src, jnp.zeros((EPAD - E,), jnp.int32)]).reshape(NW, NCHUNK, C)
    dstp = jnp.concatenate(
        [dst, jnp.full((EPAD - E,), NPAD - 1, jnp.int32)]).reshape(NW, NCHUNK, C)
    xp = jnp.pad(x, ((0, NPAD - N), (0, 0)))

    degp = _sc_degree(dstp)
    degt = degp.T

    dinv, h1p = _tc1(xp, W1, degt)
    s1 = _sc_gather_scatter(h1p, srcp, dstp)
    h2p = _tc2(s1, h1p, dinv, b1.reshape(1, D), W2)
    s2 = _sc_gather_scatter(h2p, srcp, dstp)
    return _tc3(s2, h2p, dinv, b2.reshape(1, D))


# speedup vs baseline: 8.7131x; 1.1091x over previous
import functools

import jax
import jax.numpy as jnp
from jax import lax
from jax.experimental import pallas as pl
from jax.experimental.pallas import tpu as pltpu
from jax.experimental.pallas import tpu_sc as plsc

N = 10000
E = 320000
D = 128

NC = 2
NS = 16
NW = NC * NS
NPAD = 10240
RPT = NPAD // NS
C = 128
EPAD = 327680
NCHUNK = EPAD // (NW * C)

_mesh = plsc.VectorSubcoreMesh(core_axis_name="c", subcore_axis_name="s")


@functools.partial(
    pl.kernel,
    out_type=jax.ShapeDtypeStruct((NC, NPAD), jnp.float32),
    mesh=_mesh,
    scratch_types=[
        pltpu.VMEM((2, C), jnp.int32),
        pltpu.VMEM((C,), jnp.float32),
        pltpu.VMEM((RPT,), jnp.float32),
        pltpu.VMEM_SHARED((NPAD,), jnp.float32),
        pltpu.SemaphoreType.DMA,
        pltpu.SemaphoreType.DMA,
    ],
)
def _sc_degree(dst_hbm, out_hbm, dst_v, ones_v, zero_v, deg_sh, sem_a, sem_b):
    cid = lax.axis_index("c")
    sid = lax.axis_index("s")
    wid = cid * NS + sid

    def fill(i, _):
        ones_v[pl.ds(i * 16, 16)] = jnp.ones((16,), jnp.float32)
        return 0
    lax.fori_loop(0, C // 16, fill, 0)

    def fillz(i, _):
        zero_v[pl.ds(i * 16, 16)] = jnp.zeros((16,), jnp.float32)
        return 0
    lax.fori_loop(0, RPT // 16, fillz, 0)

    pltpu.sync_copy(zero_v, deg_sh.at[pl.ds(sid * RPT, RPT)])
    plsc.subcore_barrier()

    pltpu.async_copy(dst_hbm.at[wid, 0], dst_v.at[0], sem_a)

    def body(i, _):
        j0 = 2 * i
        j1 = 2 * i + 1
        pltpu.async_copy(dst_hbm.at[wid, j1], dst_v.at[1], sem_b)
        pltpu.make_async_copy(dst_hbm.at[wid, j0], dst_v.at[0], sem_a).wait()
        pltpu.sync_copy(ones_v, deg_sh.at[dst_v.at[0]], add=True)

        @pl.when(j0 + 2 < NCHUNK)
        def _():
            pltpu.async_copy(dst_hbm.at[wid, j0 + 2], dst_v.at[0], sem_a)

        pltpu.make_async_copy(dst_hbm.at[wid, j1], dst_v.at[1], sem_b).wait()
        pltpu.sync_copy(ones_v, deg_sh.at[dst_v.at[1]], add=True)
        return 0
    lax.fori_loop(0, NCHUNK // 2, body, 0)

    plsc.subcore_barrier()
    pltpu.sync_copy(deg_sh.at[pl.ds(sid * RPT, RPT)],
                    out_hbm.at[cid, pl.ds(sid * RPT, RPT)])


@functools.partial(
    pl.kernel,
    out_type=jax.ShapeDtypeStruct((NC, NPAD, D), jnp.float32),
    mesh=_mesh,
    scratch_types=[
        pltpu.VMEM((2, C), jnp.int32),
        pltpu.VMEM((2, C), jnp.int32),
        pltpu.VMEM((C, D), jnp.float32),
        pltpu.VMEM((C, D), jnp.float32),
        pltpu.VMEM_SHARED((NPAD, D), jnp.float32),
        pltpu.SemaphoreType.DMA,
        pltpu.SemaphoreType.DMA,
        pltpu.SemaphoreType.DMA,
        pltpu.SemaphoreType.DMA,
        pltpu.SemaphoreType.DMA,
        pltpu.SemaphoreType.DMA,
    ],
)
def _sc_gather_scatter(h_hbm, src_hbm, dst_hbm, out_hbm,
                       src_v, dst_v, buf_a, buf_b, agg_sh,
                       isem_sa, isem_da, isem_sb, isem_db, gsem_a, gsem_b):
    cid = lax.axis_index("c")
    sid = lax.axis_index("s")
    wid = cid * NS + sid

    def fillz(i, _):
        for l in range(D // 16):
            buf_a[i, pl.ds(l * 16, 16)] = jnp.zeros((16,), jnp.float32)
        return 0
    lax.fori_loop(0, C, fillz, 0)
    for k in range(RPT // C):
        pltpu.sync_copy(buf_a, agg_sh.at[pl.ds(sid * RPT + k * C, C)])
    plsc.subcore_barrier()

    def fire_src(j, slot, sem):
        pltpu.async_copy(src_hbm.at[wid, j], src_v.at[slot], sem)

    def fire_dst(j, slot, sem):
        pltpu.async_copy(dst_hbm.at[wid, j], dst_v.at[slot], sem)

    def wait_src(j, slot, sem):
        pltpu.make_async_copy(src_hbm.at[wid, j], src_v.at[slot], sem).wait()

    def wait_dst(j, slot, sem):
        pltpu.make_async_copy(dst_hbm.at[wid, j], dst_v.at[slot], sem).wait()

    def fire_gather(slot, buf, sem):
        pltpu.async_copy(h_hbm.at[src_v.at[slot]], buf, sem)

    def wait_gather(slot, buf, sem):
        pltpu.make_async_copy(h_hbm.at[src_v.at[slot]], buf, sem).wait()

    fire_src(0, 0, isem_sa)
    fire_dst(0, 0, isem_da)
    fire_src(1, 1, isem_sb)
    fire_dst(1, 1, isem_db)
    wait_src(0, 0, isem_sa)
    fire_gather(0, buf_a, gsem_a)
    wait_src(1, 1, isem_sb)
    fire_gather(1, buf_b, gsem_b)

    def half(j, slot, buf, isem_s, isem_d, gsem):
        wait_gather(slot, buf, gsem)
        wait_dst(j, slot, isem_d)
        pltpu.sync_copy(buf, agg_sh.at[dst_v.at[slot]], add=True)

        @pl.when(j + 2 < NCHUNK)
        def _():
            fire_src(j + 2, slot, isem_s)
            fire_dst(j + 2, slot, isem_d)
            wait_src(j + 2, slot, isem_s)
            fire_gather(slot, buf, gsem)

    def body(i, _):
        half(2 * i, 0, buf_a, isem_sa, isem_da, gsem_a)
        half(2 * i + 1, 1, buf_b, isem_sb, isem_db, gsem_b)
        return 0
    lax.fori_loop(0, NCHUNK // 2, body, 0)

    plsc.subcore_barrier()
    pltpu.sync_copy(agg_sh.at[pl.ds(sid * RPT, RPT)],
                    out_hbm.at[cid, pl.ds(sid * RPT, RPT)])


BM = 1280


def _tc1_body(x_ref, w_ref, degt_ref, dinv_ref, hp_ref):
    deg = 1.0 + degt_ref[:, 0:1] + degt_ref[:, 1:2]
    dinv = lax.rsqrt(deg)
    dinv_ref[...] = dinv
    hp_ref[...] = jnp.dot(x_ref[...], w_ref[...],
                          preferred_element_type=jnp.float32) * dinv


def _tc2_body(s_ref, hp_ref, dinv_ref, b_ref, w_ref, hp2_ref):
    s = s_ref[0] + s_ref[1]
    dinv = dinv_ref[...]
    z = jnp.maximum(dinv * (s + hp_ref[...]) + b_ref[...], 0.0)
    hp2_ref[...] = jnp.dot(z, w_ref[...],
                           preferred_element_type=jnp.float32) * dinv


def _tc3_body(s_ref, hp_ref, dinv_ref, b_ref, out_ref):
    i = pl.program_id(0)
    s = s_ref[0] + s_ref[1]
    z = jnp.maximum(dinv_ref[...] * (s + hp_ref[...]) + b_ref[...], 0.0)
    rows = lax.broadcasted_iota(jnp.int32, (BM, D), 0) + i * BM
    z = jnp.where(rows < N, z, 0.0)

    @pl.when(i == 0)
    def _():
        out_ref[...] = jnp.zeros_like(out_ref)

    out_ref[...] += jnp.sum(z, axis=0, keepdims=True) * (1.0 / N)


def _tc1(xp, w1, degt):
    return pl.pallas_call(
        _tc1_body,
        grid=(NPAD // BM,),
        in_specs=[
            pl.BlockSpec((BM, D), lambda i: (i, 0)),
            pl.BlockSpec((D, D), lambda i: (0, 0)),
            pl.BlockSpec((BM, NC), lambda i: (i, 0)),
        ],
        out_specs=[
            pl.BlockSpec((BM, 1), lambda i: (i, 0)),
            pl.BlockSpec((BM, D), lambda i: (i, 0)),
        ],
        out_shape=[
            jax.ShapeDtypeStruct((NPAD, 1), jnp.float32),
            jax.ShapeDtypeStruct((NPAD, D), jnp.float32),
        ],
    )(xp, w1, degt)


def _tc2(s, hp, dinv, b, w):
    return pl.pallas_call(
        _tc2_body,
        grid=(NPAD // BM,),
        in_specs=[
            pl.BlockSpec((NC, BM, D), lambda i: (0, i, 0)),
            pl.BlockSpec((BM, D), lambda i: (i, 0)),
            pl.BlockSpec((BM, 1), lambda i: (i, 0)),
            pl.BlockSpec((1, D), lambda i: (0, 0)),
            pl.BlockSpec((D, D), lambda i: (0, 0)),
        ],
        out_specs=pl.BlockSpec((BM, D), lambda i: (i, 0)),
        out_shape=jax.ShapeDtypeStruct((NPAD, D), jnp.float32),
    )(s, hp, dinv, b, w)


def _tc3(s, hp, dinv, b):
    return pl.pallas_call(
        _tc3_body,
        grid=(NPAD // BM,),
        in_specs=[
            pl.BlockSpec((NC, BM, D), lambda i: (0, i, 0)),
            pl.BlockSpec((BM, D), lambda i: (i, 0)),
            pl.BlockSpec((BM, 1), lambda i: (i, 0)),
            pl.BlockSpec((1, D), lambda i: (0, 0)),
        ],
        out_specs=pl.BlockSpec((1, D), lambda i: (0, 0)),
        out_shape=jax.ShapeDtypeStruct((1, D), jnp.float32),
    )(s, hp, dinv, b)


def kernel(x, edge_index, W1, b1, W2, b2):
    src = edge_index[0]
    dst = edge_index[1]
    srcp = jnp.concatenate(
        [src, jnp.zeros((EPAD - E,), jnp.int32)]).reshape(NW, NCHUNK, C)
    pad_dst = N + jnp.arange(EPAD - E, dtype=jnp.int32) % (NPAD - N)
    dstp = jnp.concatenate([dst, pad_dst]).reshape(NW, NCHUNK, C)
    xp = jnp.pad(x, ((0, NPAD - N), (0, 0)))

    degp = _sc_degree(dstp)
    degt = degp.T

    dinv, h1p = _tc1(xp, W1, degt)
    s1 = _sc_gather_scatter(h1p, srcp, dstp)
    h2p = _tc2(s1, h1p, dinv, b1.reshape(1, D), W2)
    s2 = _sc_gather_scatter(h2p, srcp, dstp)
    return _tc3(s2, h2p, dinv, b2.reshape(1, D))


# speedup vs baseline: 25.7754x; 2.9582x over previous
import functools

import jax
import jax.numpy as jnp
from jax import lax
from jax.experimental import pallas as pl
from jax.experimental.pallas import tpu as pltpu
from jax.experimental.pallas import tpu_sc as plsc

N = 10000
E = 320000
D = 128

NC = 2
NS = 16
NW = NC * NS
NPAD = 10240
RPT = NPAD // NS
C = 128
EPAD = 327680
NCHUNK = EPAD // (NW * C)

_mesh = plsc.VectorSubcoreMesh(core_axis_name="c", subcore_axis_name="s")


@functools.partial(
    pl.kernel,
    out_type=jax.ShapeDtypeStruct((NC, NPAD), jnp.float32),
    mesh=_mesh,
    scratch_types=[
        pltpu.VMEM((2, C), jnp.int32),
        pltpu.VMEM((C,), jnp.float32),
        pltpu.VMEM((RPT,), jnp.float32),
        pltpu.VMEM_SHARED((NPAD,), jnp.float32),
        pltpu.SemaphoreType.DMA,
        pltpu.SemaphoreType.DMA,
    ],
)
def _sc_degree(dst_hbm, out_hbm, dst_v, ones_v, zero_v, deg_sh, sem_a, sem_b):
    cid = lax.axis_index("c")
    sid = lax.axis_index("s")
    wid = cid * NS + sid

    def fill(i, _):
        ones_v[pl.ds(i * 16, 16)] = jnp.ones((16,), jnp.float32)
        return 0
    lax.fori_loop(0, C // 16, fill, 0)

    def fillz(i, _):
        zero_v[pl.ds(i * 16, 16)] = jnp.zeros((16,), jnp.float32)
        return 0
    lax.fori_loop(0, RPT // 16, fillz, 0)

    pltpu.sync_copy(zero_v, deg_sh.at[pl.ds(sid * RPT, RPT)])
    plsc.subcore_barrier()

    pltpu.async_copy(dst_hbm.at[wid, 0], dst_v.at[0], sem_a)

    def body(i, _):
        j0 = 2 * i
        j1 = 2 * i + 1
        pltpu.async_copy(dst_hbm.at[wid, j1], dst_v.at[1], sem_b)
        pltpu.make_async_copy(dst_hbm.at[wid, j0], dst_v.at[0], sem_a).wait()
        pltpu.sync_copy(ones_v, deg_sh.at[dst_v.at[0]], add=True)

        @pl.when(j0 + 2 < NCHUNK)
        def _():
            pltpu.async_copy(dst_hbm.at[wid, j0 + 2], dst_v.at[0], sem_a)

        pltpu.make_async_copy(dst_hbm.at[wid, j1], dst_v.at[1], sem_b).wait()
        pltpu.sync_copy(ones_v, deg_sh.at[dst_v.at[1]], add=True)
        return 0
    lax.fori_loop(0, NCHUNK // 2, body, 0)

    plsc.subcore_barrier()
    pltpu.sync_copy(deg_sh.at[pl.ds(sid * RPT, RPT)],
                    out_hbm.at[cid, pl.ds(sid * RPT, RPT)])


@functools.partial(
    pl.kernel,
    out_type=jax.ShapeDtypeStruct((NC, NPAD, D), jnp.float32),
    mesh=_mesh,
    scratch_types=[
        pltpu.VMEM((2, C), jnp.int32),
        pltpu.VMEM((2, C), jnp.int32),
        pltpu.VMEM((C, D), jnp.float32),
        pltpu.VMEM((C, D), jnp.float32),
        pltpu.VMEM_SHARED((NPAD, D), jnp.float32),
        pltpu.SemaphoreType.DMA,
        pltpu.SemaphoreType.DMA,
        pltpu.SemaphoreType.DMA,
        pltpu.SemaphoreType.DMA,
        pltpu.SemaphoreType.DMA,
        pltpu.SemaphoreType.DMA,
    ],
)
def _sc_gather_scatter(h_hbm, src_hbm, dst_hbm, out_hbm,
                       src_v, dst_v, buf_a, buf_b, agg_sh,
                       isem_sa, isem_da, isem_sb, isem_db, gsem_a, gsem_b):
    cid = lax.axis_index("c")
    sid = lax.axis_index("s")
    wid = cid * NS + sid

    def fillz(i, _):
        for l in range(D // 16):
            buf_a[i, pl.ds(l * 16, 16)] = jnp.zeros((16,), jnp.float32)
        return 0
    lax.fori_loop(0, C, fillz, 0)
    for k in range(RPT // C):
        pltpu.sync_copy(buf_a, agg_sh.at[pl.ds(sid * RPT + k * C, C)])
    plsc.subcore_barrier()

    def fire_src(j, slot, sem):
        pltpu.async_copy(src_hbm.at[wid, j], src_v.at[slot], sem)

    def fire_dst(j, slot, sem):
        pltpu.async_copy(dst_hbm.at[wid, j], dst_v.at[slot], sem)

    def wait_src(j, slot, sem):
        pltpu.make_async_copy(src_hbm.at[wid, j], src_v.at[slot], sem).wait()

    def wait_dst(j, slot, sem):
        pltpu.make_async_copy(dst_hbm.at[wid, j], dst_v.at[slot], sem).wait()

    def fire_gather(slot, buf, sem):
        pltpu.async_copy(h_hbm.at[src_v.at[slot]], buf, sem)

    def wait_gather(slot, buf, sem):
        pltpu.make_async_copy(h_hbm.at[src_v.at[slot]], buf, sem).wait()

    fire_src(0, 0, isem_sa)
    fire_dst(0, 0, isem_da)
    fire_src(1, 1, isem_sb)
    fire_dst(1, 1, isem_db)
    wait_src(0, 0, isem_sa)
    fire_gather(0, buf_a, gsem_a)
    wait_src(1, 1, isem_sb)
    fire_gather(1, buf_b, gsem_b)

    def half(j, slot, buf, isem_s, isem_d, gsem):
        wait_gather(slot, buf, gsem)
        wait_dst(j, slot, isem_d)
        pltpu.sync_copy(buf, agg_sh.at[dst_v.at[slot]], add=True)

        @pl.when(j + 2 < NCHUNK)
        def _():
            fire_src(j + 2, slot, isem_s)
            fire_dst(j + 2, slot, isem_d)
            wait_src(j + 2, slot, isem_s)
            fire_gather(slot, buf, gsem)

    def body(i, _):
        half(2 * i, 0, buf_a, isem_sa, isem_da, gsem_a)
        half(2 * i + 1, 1, buf_b, isem_sb, isem_db, gsem_b)
        return 0
    lax.fori_loop(0, NCHUNK // 2, body, 0)

    plsc.subcore_barrier()
    pltpu.sync_copy(agg_sh.at[pl.ds(sid * RPT, RPT)],
                    out_hbm.at[cid, pl.ds(sid * RPT, RPT)])


BM = 1280


def _tc1_body(x_ref, w_ref, degt_ref, dinv_ref, hp_ref):
    deg = 1.0 + degt_ref[:, 0:1] + degt_ref[:, 1:2]
    dinv = lax.rsqrt(deg)
    dinv_ref[...] = dinv
    hp_ref[...] = jnp.dot(x_ref[...], w_ref[...],
                          preferred_element_type=jnp.float32) * dinv


def _tc2_body(s_ref, hp_ref, dinv_ref, b_ref, w_ref, hp2_ref):
    s = s_ref[0] + s_ref[1]
    dinv = dinv_ref[...]
    z = jnp.maximum(dinv * (s + hp_ref[...]) + b_ref[...], 0.0)
    hp2_ref[...] = jnp.dot(z, w_ref[...],
                           preferred_element_type=jnp.float32) * dinv


def _tc3_body(s_ref, hp_ref, dinv_ref, b_ref, out_ref):
    i = pl.program_id(0)
    s = s_ref[0] + s_ref[1]
    z = jnp.maximum(dinv_ref[...] * (s + hp_ref[...]) + b_ref[...], 0.0)
    rows = lax.broadcasted_iota(jnp.int32, (BM, D), 0) + i * BM
    z = jnp.where(rows < N, z, 0.0)

    @pl.when(i == 0)
    def _():
        out_ref[...] = jnp.zeros_like(out_ref)

    out_ref[...] += jnp.sum(z, axis=0, keepdims=True) * (1.0 / N)


def _tc1(xp, w1, degt):
    return pl.pallas_call(
        _tc1_body,
        grid=(NPAD // BM,),
        in_specs=[
            pl.BlockSpec((BM, D), lambda i: (i, 0)),
            pl.BlockSpec((D, D), lambda i: (0, 0)),
            pl.BlockSpec((BM, NC), lambda i: (i, 0)),
        ],
        out_specs=[
            pl.BlockSpec((BM, 1), lambda i: (i, 0)),
            pl.BlockSpec((BM, D), lambda i: (i, 0)),
        ],
        out_shape=[
            jax.ShapeDtypeStruct((NPAD, 1), jnp.float32),
            jax.ShapeDtypeStruct((NPAD, D), jnp.float32),
        ],
    )(xp, w1, degt)


def _tc2(s, hp, dinv, b, w):
    return pl.pallas_call(
        _tc2_body,
        grid=(NPAD // BM,),
        in_specs=[
            pl.BlockSpec((NC, BM, D), lambda i: (0, i, 0)),
            pl.BlockSpec((BM, D), lambda i: (i, 0)),
            pl.BlockSpec((BM, 1), lambda i: (i, 0)),
            pl.BlockSpec((1, D), lambda i: (0, 0)),
            pl.BlockSpec((D, D), lambda i: (0, 0)),
        ],
        out_specs=pl.BlockSpec((BM, D), lambda i: (i, 0)),
        out_shape=jax.ShapeDtypeStruct((NPAD, D), jnp.float32),
    )(s, hp, dinv, b, w)


def _tc3(s, hp, dinv, b):
    return pl.pallas_call(
        _tc3_body,
        grid=(NPAD // BM,),
        in_specs=[
            pl.BlockSpec((NC, BM, D), lambda i: (0, i, 0)),
            pl.BlockSpec((BM, D), lambda i: (i, 0)),
            pl.BlockSpec((BM, 1), lambda i: (i, 0)),
            pl.BlockSpec((1, D), lambda i: (0, 0)),
        ],
        out_specs=pl.BlockSpec((1, D), lambda i: (0, 0)),
        out_shape=jax.ShapeDtypeStruct((1, D), jnp.float32),
    )(s, hp, dinv, b)


def kernel(x, edge_index, W1, b1, W2, b2):
    src = edge_index[0]
    dst = edge_index[1]
    pad_src = jnp.arange(EPAD - E, dtype=jnp.int32) * 13 % N
    srcp = jnp.concatenate([src, pad_src]).reshape(NW, NCHUNK, C)
    pad_dst = N + jnp.arange(EPAD - E, dtype=jnp.int32) % (NPAD - N)
    dstp = jnp.concatenate([dst, pad_dst]).reshape(NW, NCHUNK, C)
    xp = jnp.pad(x, ((0, NPAD - N), (0, 0)))

    degp = _sc_degree(dstp)
    degt = degp.T

    dinv, h1p = _tc1(xp, W1, degt)
    s1 = _sc_gather_scatter(h1p, srcp, dstp)
    h2p = _tc2(s1, h1p, dinv, b1.reshape(1, D), W2)
    s2 = _sc_gather_scatter(h2p, srcp, dstp)
    return _tc3(s2, h2p, dinv, b2.reshape(1, D))


# speedup vs baseline: 26.8408x; 1.0413x over previous
import functools

import jax
import jax.numpy as jnp
from jax import lax
from jax.experimental import pallas as pl
from jax.experimental.pallas import tpu as pltpu
from jax.experimental.pallas import tpu_sc as plsc

N = 10000
E = 320000
D = 128

NC = 2
NS = 16
NW = NC * NS
NPAD = 10240
RPT = NPAD // NS
EPT = E // NW
C = 128
NF = EPT // C
TAIL = EPT - NF * C

_mesh = plsc.VectorSubcoreMesh(core_axis_name="c", subcore_axis_name="s")


@functools.partial(
    pl.kernel,
    out_type=jax.ShapeDtypeStruct((NC, NPAD), jnp.float32),
    mesh=_mesh,
    scratch_types=[
        pltpu.VMEM((2, C), jnp.int32),
        pltpu.VMEM((TAIL,), jnp.int32),
        pltpu.VMEM((C,), jnp.float32),
        pltpu.VMEM((RPT,), jnp.float32),
        pltpu.VMEM_SHARED((NPAD,), jnp.float32),
        pltpu.SemaphoreType.DMA,
        pltpu.SemaphoreType.DMA,
    ],
)
def _sc_degree(edges_hbm, out_hbm, dst_v, tdst_v, ones_v, zero_v, deg_sh,
               sem_a, sem_b):
    cid = lax.axis_index("c")
    sid = lax.axis_index("s")
    wid = cid * NS + sid

    def fill(i, _):
        ones_v[pl.ds(i * 16, 16)] = jnp.ones((16,), jnp.float32)
        return 0
    lax.fori_loop(0, C // 16, fill, 0)

    def fillz(i, _):
        zero_v[pl.ds(i * 16, 16)] = jnp.zeros((16,), jnp.float32)
        return 0
    lax.fori_loop(0, RPT // 16, fillz, 0)

    pltpu.sync_copy(zero_v, deg_sh.at[pl.ds(sid * RPT, RPT)])
    plsc.subcore_barrier()

    def dst_src_ref(j):
        return edges_hbm.at[1, wid, pl.ds(j * C, C)]

    pltpu.async_copy(dst_src_ref(0), dst_v.at[0], sem_a)

    def body(i, _):
        j0 = 2 * i
        j1 = 2 * i + 1
        pltpu.async_copy(dst_src_ref(j1), dst_v.at[1], sem_b)
        pltpu.make_async_copy(dst_src_ref(j0), dst_v.at[0], sem_a).wait()
        pltpu.sync_copy(ones_v, deg_sh.at[dst_v.at[0]], add=True)

        @pl.when(j0 + 2 < NF)
        def _():
            pltpu.async_copy(dst_src_ref(j0 + 2), dst_v.at[0], sem_a)

        pltpu.make_async_copy(dst_src_ref(j1), dst_v.at[1], sem_b).wait()
        pltpu.sync_copy(ones_v, deg_sh.at[dst_v.at[1]], add=True)
        return 0
    lax.fori_loop(0, NF // 2, body, 0)

    pltpu.sync_copy(edges_hbm.at[1, wid, pl.ds(NF * C, TAIL)], tdst_v)
    pltpu.sync_copy(ones_v.at[pl.ds(0, TAIL)], deg_sh.at[tdst_v], add=True)

    plsc.subcore_barrier()
    pltpu.sync_copy(deg_sh.at[pl.ds(sid * RPT, RPT)],
                    out_hbm.at[cid, pl.ds(sid * RPT, RPT)])


@functools.partial(
    pl.kernel,
    out_type=jax.ShapeDtypeStruct((NC, NPAD, D), jnp.float32),
    mesh=_mesh,
    scratch_types=[
        pltpu.VMEM((2, C), jnp.int32),
        pltpu.VMEM((2, C), jnp.int32),
        pltpu.VMEM((TAIL,), jnp.int32),
        pltpu.VMEM((TAIL,), jnp.int32),
        pltpu.VMEM((C, D), jnp.float32),
        pltpu.VMEM((C, D), jnp.float32),
        pltpu.VMEM_SHARED((NPAD, D), jnp.float32),
        pltpu.SemaphoreType.DMA,
        pltpu.SemaphoreType.DMA,
        pltpu.SemaphoreType.DMA,
        pltpu.SemaphoreType.DMA,
        pltpu.SemaphoreType.DMA,
        pltpu.SemaphoreType.DMA,
    ],
)
def _sc_gather_scatter(h_hbm, edges_hbm, out_hbm,
                       src_v, dst_v, tsrc_v, tdst_v, buf_a, buf_b, agg_sh,
                       isem_sa, isem_da, isem_sb, isem_db, gsem_a, gsem_b):
    cid = lax.axis_index("c")
    sid = lax.axis_index("s")
    wid = cid * NS + sid

    def fillz(i, _):
        for l in range(D // 16):
            buf_a[i, pl.ds(l * 16, 16)] = jnp.zeros((16,), jnp.float32)
        return 0
    lax.fori_loop(0, C, fillz, 0)
    for k in range(RPT // C):
        pltpu.sync_copy(buf_a, agg_sh.at[pl.ds(sid * RPT + k * C, C)])
    plsc.subcore_barrier()

    def src_src_ref(j):
        return edges_hbm.at[0, wid, pl.ds(j * C, C)]

    def dst_src_ref(j):
        return edges_hbm.at[1, wid, pl.ds(j * C, C)]

    def fire_src(j, slot, sem):
        pltpu.async_copy(src_src_ref(j), src_v.at[slot], sem)

    def fire_dst(j, slot, sem):
        pltpu.async_copy(dst_src_ref(j), dst_v.at[slot], sem)

    def wait_src(j, slot, sem):
        pltpu.make_async_copy(src_src_ref(j), src_v.at[slot], sem).wait()

    def wait_dst(j, slot, sem):
        pltpu.make_async_copy(dst_src_ref(j), dst_v.at[slot], sem).wait()

    def fire_gather(slot, buf, sem):
        pltpu.async_copy(h_hbm.at[src_v.at[slot]], buf, sem)

    def wait_gather(slot, buf, sem):
        pltpu.make_async_copy(h_hbm.at[src_v.at[slot]], buf, sem).wait()

    fire_src(0, 0, isem_sa)
    fire_dst(0, 0, isem_da)
    fire_src(1, 1, isem_sb)
    fire_dst(1, 1, isem_db)
    wait_src(0, 0, isem_sa)
    fire_gather(0, buf_a, gsem_a)
    wait_src(1, 1, isem_sb)
    fire_gather(1, buf_b, gsem_b)

    def half(j, slot, buf, isem_s, isem_d, gsem):
        wait_gather(slot, buf, gsem)
        wait_dst(j, slot, isem_d)
        pltpu.sync_copy(buf, agg_sh.at[dst_v.at[slot]], add=True)

        @pl.when(j + 2 < NF)
        def _():
            fire_src(j + 2, slot, isem_s)
            fire_dst(j + 2, slot, isem_d)
            wait_src(j + 2, slot, isem_s)
            fire_gather(slot, buf, gsem)

    def body(i, _):
        half(2 * i, 0, buf_a, isem_sa, isem_da, gsem_a)
        half(2 * i + 1, 1, buf_b, isem_sb, isem_db, gsem_b)
        return 0
    lax.fori_loop(0, NF // 2, body, 0)

    pltpu.sync_copy(edges_hbm.at[0, wid, pl.ds(NF * C, TAIL)], tsrc_v)
    pltpu.sync_copy(edges_hbm.at[1, wid, pl.ds(NF * C, TAIL)], tdst_v)
    pltpu.async_copy(h_hbm.at[tsrc_v], buf_a.at[pl.ds(0, TAIL)], gsem_a).wait()
    pltpu.sync_copy(buf_a.at[pl.ds(0, TAIL)], agg_sh.at[tdst_v], add=True)

    plsc.subcore_barrier()
    pltpu.sync_copy(agg_sh.at[pl.ds(sid * RPT, RPT)],
                    out_hbm.at[cid, pl.ds(sid * RPT, RPT)])


BM = 2000


def _tc_mm_body(x_ref, w_ref, h_ref):
    h_ref[...] = jnp.dot(x_ref[...], w_ref[...],
                         preferred_element_type=jnp.float32)


def _tc_scale_body(h_ref, degt_ref, dinv_ref, hp_ref):
    deg = 1.0 + degt_ref[:, 0:1] + degt_ref[:, 1:2]
    dinv = lax.rsqrt(deg)
    dinv_ref[...] = dinv
    hp_ref[...] = h_ref[...] * dinv


def _tc2_body(s_ref, hp_ref, dinv_ref, b_ref, w_ref, hp2_ref):
    s = s_ref[0] + s_ref[1]
    dinv = dinv_ref[...]
    z = jnp.maximum(dinv * (s + hp_ref[...]) + b_ref[...], 0.0)
    hp2_ref[...] = jnp.dot(z, w_ref[...],
                           preferred_element_type=jnp.float32) * dinv


def _tc3_body(s_ref, hp_ref, dinv_ref, b_ref, out_ref):
    i = pl.program_id(0)
    s = s_ref[0] + s_ref[1]
    z = jnp.maximum(dinv_ref[...] * (s + hp_ref[...]) + b_ref[...], 0.0)

    @pl.when(i == 0)
    def _():
        out_ref[...] = jnp.zeros_like(out_ref)

    out_ref[...] += jnp.sum(z, axis=0, keepdims=True) * (1.0 / N)


def _tc_mm(x, w):
    return pl.pallas_call(
        _tc_mm_body,
        grid=(N // BM,),
        in_specs=[
            pl.BlockSpec((BM, D), lambda i: (i, 0)),
            pl.BlockSpec((D, D), lambda i: (0, 0)),
        ],
        out_specs=pl.BlockSpec((BM, D), lambda i: (i, 0)),
        out_shape=jax.ShapeDtypeStruct((N, D), jnp.float32),
    )(x, w)


def _tc_scale(h, degt):
    return pl.pallas_call(
        _tc_scale_body,
        grid=(N // BM,),
        in_specs=[
            pl.BlockSpec((BM, D), lambda i: (i, 0)),
            pl.BlockSpec((BM, NC), lambda i: (i, 0)),
        ],
        out_specs=[
            pl.BlockSpec((BM, 1), lambda i: (i, 0)),
            pl.BlockSpec((BM, D), lambda i: (i, 0)),
        ],
        out_shape=[
            jax.ShapeDtypeStruct((N, 1), jnp.float32),
            jax.ShapeDtypeStruct((N, D), jnp.float32),
        ],
    )(h, degt)


def _tc2(s, hp, dinv, b, w):
    return pl.pallas_call(
        _tc2_body,
        grid=(N // BM,),
        in_specs=[
            pl.BlockSpec((NC, BM, D), lambda i: (0, i, 0)),
            pl.BlockSpec((BM, D), lambda i: (i, 0)),
            pl.BlockSpec((BM, 1), lambda i: (i, 0)),
            pl.BlockSpec((1, D), lambda i: (0, 0)),
            pl.BlockSpec((D, D), lambda i: (0, 0)),
        ],
        out_specs=pl.BlockSpec((BM, D), lambda i: (i, 0)),
        out_shape=jax.ShapeDtypeStruct((N, D), jnp.float32),
    )(s, hp, dinv, b, w)


def _tc3(s, hp, dinv, b):
    return pl.pallas_call(
        _tc3_body,
        grid=(N // BM,),
        in_specs=[
            pl.BlockSpec((NC, BM, D), lambda i: (0, i, 0)),
            pl.BlockSpec((BM, D), lambda i: (i, 0)),
            pl.BlockSpec((BM, 1), lambda i: (i, 0)),
            pl.BlockSpec((1, D), lambda i: (0, 0)),
        ],
        out_specs=pl.BlockSpec((1, D), lambda i: (0, 0)),
        out_shape=jax.ShapeDtypeStruct((1, D), jnp.float32),
    )(s, hp, dinv, b)


def kernel(x, edge_index, W1, b1, W2, b2):
    edges = edge_index.reshape(2, NW, EPT)

    degp = _sc_degree(edges)
    h1 = _tc_mm(x, W1)
    degt = degp.T[:N]

    dinv, h1p = _tc_scale(h1, degt)
    s1 = _sc_gather_scatter(h1p, edges)
    h2p = _tc2(s1, h1p, dinv, b1.reshape(1, D), W2)
    s2 = _sc_gather_scatter(h2p, edges)
    return _tc3(s2, h2p, dinv, b2.reshape(1, D))
